# baseline (speedup 1.0000x reference)
import functools
import math

import jax
import jax.numpy as jnp
from jax import lax
from jax.experimental import pallas as pl
from jax.experimental.pallas import tpu as pltpu
from jax.experimental.pallas import tpu_sc as plsc

DIM = 1024
THETA = 2000.0
HALF = DIM // 2
EMBED_SCALE = math.sqrt(DIM)
LN_THETA = math.log(THETA)
HALF_PI = math.pi / 2.0

_SC_NC = 2
_SC_NS = 16
_SC_NW = _SC_NC * _SC_NS

_CH = 32


def _sc_gather(table, idx, n_tok):
    per_w = n_tok // _SC_NW
    n_chunks = per_w // _CH
    mesh = plsc.VectorSubcoreMesh(core_axis_name="c", subcore_axis_name="s")

    @functools.partial(
        pl.kernel,
        mesh=mesh,
        out_type=jax.ShapeDtypeStruct((n_tok, DIM), jnp.float32),
        scratch_types=[
            pltpu.VMEM((per_w,), jnp.int32),
            pltpu.VMEM((_CH, DIM), jnp.float32),
            pltpu.VMEM((_CH, DIM), jnp.float32),
            pltpu.SemaphoreType.DMA,
            pltpu.SemaphoreType.DMA,
        ],
    )
    def k(table_hbm, idx_hbm, out_hbm, idx_v, buf0, buf1, sem0, sem1):
        wid = lax.axis_index("s") * _SC_NC + lax.axis_index("c")
        base = wid * per_w
        pltpu.sync_copy(idx_hbm.at[pl.ds(base, per_w)], idx_v)
        bufs = (buf0, buf1)
        sems = (sem0, sem1)
        pltpu.async_copy(table_hbm.at[idx_v.at[pl.ds(0, _CH)]], buf0, sem0)
        for c in range(n_chunks):
            cur = c % 2
            nxt = (c + 1) % 2
            if c + 1 < n_chunks:
                pltpu.async_copy(
                    table_hbm.at[idx_v.at[pl.ds((c + 1) * _CH, _CH)]],
                    bufs[nxt], sems[nxt])
            pltpu.make_async_copy(
                table_hbm.at[idx_v.at[pl.ds(c * _CH, _CH)]],
                bufs[cur], sems[cur]).wait()
            pltpu.sync_copy(bufs[cur], out_hbm.at[pl.ds(base + c * _CH, _CH)])

    return k(table, idx)


def _tc_finish(raw, scale, n_tok, seq_len):
    bs = 256
    n_seq_blocks = seq_len // bs
    n_batch = n_tok // seq_len

    def body(scale_ref, raw_ref, x_ref, emb_ref, pos_vmem):
        i = pl.program_id(0)
        j = pl.program_id(1)

        @pl.when(j == 0)
        def _():
            p = (lax.broadcasted_iota(jnp.int32, (bs, DIM), 0)
                 + i * bs).astype(jnp.float32)
            col = lax.broadcasted_iota(jnp.int32, (bs, DIM), 1)
            is_cos = col >= HALF
            f = jnp.where(is_cos, col - HALF, col).astype(jnp.float32)
            inv_freq = jnp.exp(f * (-LN_THETA / HALF))
            arg = p * inv_freq + jnp.where(is_cos, HALF_PI, 0.0)
            pos_vmem[...] = jnp.sin(arg) * scale_ref[0, 0]

        e = raw_ref[...] * EMBED_SCALE
        emb_ref[...] = e
        x_ref[...] = e + pos_vmem[...]

    grid = (n_seq_blocks, n_batch)
    blk = pl.BlockSpec((bs, DIM), lambda i, j: (j * n_seq_blocks + i, 0))
    return pl.pallas_call(
        body,
        grid=grid,
        in_specs=[
            pl.BlockSpec((1, 1), lambda i, j: (0, 0),
                         memory_space=pltpu.SMEM),
            blk,
        ],
        out_specs=(blk, blk),
        out_shape=(
            jax.ShapeDtypeStruct((n_tok, DIM), jnp.float32),
            jax.ShapeDtypeStruct((n_tok, DIM), jnp.float32),
        ),
        scratch_shapes=[pltpu.VMEM((bs, DIM), jnp.float32)],
    )(scale.reshape(1, 1), raw)


def kernel(src_tokens, table, scale):
    n_batch, seq_len = src_tokens.shape
    n_tok = n_batch * seq_len
    idx = src_tokens.reshape(-1).astype(jnp.int32)
    raw = _sc_gather(table, idx, n_tok)
    x, embed = _tc_finish(raw, scale, n_tok, seq_len)
    out_shape = (n_batch, seq_len, DIM)
    return (x.reshape(out_shape), embed.reshape(out_shape))


# speedup vs baseline: 1.0006x; 1.0006x over previous
import functools
import math

import jax
import jax.numpy as jnp
from jax import lax
from jax.experimental import pallas as pl
from jax.experimental.pallas import tpu as pltpu
from jax.experimental.pallas import tpu_sc as plsc

DIM = 1024
THETA = 2000.0
HALF = DIM // 2
EMBED_SCALE = math.sqrt(DIM)
LN_THETA = math.log(THETA)
HALF_PI = math.pi / 2.0

_SC_NC = 2
_SC_NS = 16
_SC_NW = _SC_NC * _SC_NS

_CH = 32


def _sc_gather(table, idx, n_tok):
    per_w = n_tok // _SC_NW
    n_chunks = per_w // _CH
    mesh = plsc.VectorSubcoreMesh(core_axis_name="c", subcore_axis_name="s")

    @functools.partial(
        pl.kernel,
        mesh=mesh,
        out_type=jax.ShapeDtypeStruct((n_tok, DIM), jnp.float32),
        scratch_types=[
            pltpu.VMEM((per_w,), jnp.int32),
            pltpu.VMEM((_CH, DIM), jnp.float32),
            pltpu.VMEM((_CH, DIM), jnp.float32),
            pltpu.SemaphoreType.DMA,
            pltpu.SemaphoreType.DMA,
        ],
    )
    def k(table_hbm, idx_hbm, out_hbm, idx_v, buf0, buf1, sem0, sem1):
        wid = lax.axis_index("s") * _SC_NC + lax.axis_index("c")
        base = wid * per_w
        pltpu.sync_copy(idx_hbm.at[pl.ds(base, per_w)], idx_v)
        bufs = (buf0, buf1)
        sems = (sem0, sem1)
        pltpu.async_copy(table_hbm.at[idx_v.at[pl.ds(0, _CH)]], buf0, sem0)
        for c in range(n_chunks):
            cur = c % 2
            nxt = (c + 1) % 2
            if c + 1 < n_chunks:
                pltpu.async_copy(
                    table_hbm.at[idx_v.at[pl.ds((c + 1) * _CH, _CH)]],
                    bufs[nxt], sems[nxt])
            pltpu.make_async_copy(
                table_hbm.at[idx_v.at[pl.ds(c * _CH, _CH)]],
                bufs[cur], sems[cur]).wait()
            pltpu.sync_copy(bufs[cur], out_hbm.at[pl.ds(base + c * _CH, _CH)])

    return k(table, idx)


def _tc_finish(raw, scale, n_tok, seq_len):
    bs = 256
    n_seq_blocks = seq_len // bs
    n_batch = n_tok // seq_len

    def body(scale_ref, raw_ref, x_ref, emb_ref, pos_vmem):
        i = pl.program_id(0)
        j = pl.program_id(1)

        @pl.when(j == 0)
        def _():
            p = (lax.broadcasted_iota(jnp.int32, (bs, DIM), 0)
                 + i * bs).astype(jnp.float32)
            col = lax.broadcasted_iota(jnp.int32, (1, DIM), 1)
            is_cos = col >= HALF
            f = jnp.where(is_cos, col - HALF, col).astype(jnp.float32)
            inv_freq = jnp.exp(f * (-LN_THETA / HALF))
            shift = jnp.where(is_cos, HALF_PI, 0.0)
            arg = p * inv_freq + shift
            pos_vmem[...] = jnp.sin(arg) * scale_ref[0, 0]

        e = raw_ref[...] * EMBED_SCALE
        emb_ref[...] = e
        x_ref[...] = e + pos_vmem[...]

    grid = (n_seq_blocks, n_batch)
    blk = pl.BlockSpec((bs, DIM), lambda i, j: (j * n_seq_blocks + i, 0))
    return pl.pallas_call(
        body,
        grid=grid,
        in_specs=[
            pl.BlockSpec((1, 1), lambda i, j: (0, 0),
                         memory_space=pltpu.SMEM),
            blk,
        ],
        out_specs=(blk, blk),
        out_shape=(
            jax.ShapeDtypeStruct((n_tok, DIM), jnp.float32),
            jax.ShapeDtypeStruct((n_tok, DIM), jnp.float32),
        ),
        scratch_shapes=[pltpu.VMEM((bs, DIM), jnp.float32)],
    )(scale.reshape(1, 1), raw)


def kernel(src_tokens, table, scale):
    n_batch, seq_len = src_tokens.shape
    n_tok = n_batch * seq_len
    idx = src_tokens.reshape(-1).astype(jnp.int32)
    raw = _sc_gather(table, idx, n_tok)
    x, embed = _tc_finish(raw, scale, n_tok, seq_len)
    out_shape = (n_batch, seq_len, DIM)
    return (x.reshape(out_shape), embed.reshape(out_shape))


# speedup vs baseline: 1.2672x; 1.2664x over previous
import functools
import math

import jax
import jax.numpy as jnp
from jax import lax
from jax.experimental import pallas as pl
from jax.experimental.pallas import tpu as pltpu
from jax.experimental.pallas import tpu_sc as plsc

DIM = 1024
THETA = 2000.0
HALF = DIM // 2
EMBED_SCALE = math.sqrt(DIM)
LN_THETA = math.log(THETA)
HALF_PI = math.pi / 2.0

_SC_NC = 2
_SC_NS = 16
_SC_NW = _SC_NC * _SC_NS

_CH = 32
_NBUF = 3


def _sc_gather_scale(table, idx, n_tok):
    per_w = n_tok // _SC_NW
    n_chunks = per_w // _CH
    mesh = plsc.VectorSubcoreMesh(core_axis_name="c", subcore_axis_name="s")

    scratch = [pltpu.VMEM((per_w,), jnp.int32)]
    scratch += [pltpu.VMEM((_CH, DIM), jnp.float32) for _ in range(_NBUF)]
    scratch += [pltpu.SemaphoreType.DMA for _ in range(2 * _NBUF)]

    @functools.partial(
        pl.kernel,
        mesh=mesh,
        out_type=jax.ShapeDtypeStruct((n_tok, DIM), jnp.float32),
        scratch_types=scratch,
    )
    def k(table_hbm, idx_hbm, out_hbm, idx_v, *bufs_sems):
        bufs = bufs_sems[:_NBUF]
        gsem = bufs_sems[_NBUF:2 * _NBUF]
        wsem = bufs_sems[2 * _NBUF:]
        wid = lax.axis_index("s") * _SC_NC + lax.axis_index("c")
        base = wid * per_w
        pltpu.sync_copy(idx_hbm.at[pl.ds(base, per_w)], idx_v)

        def gather_start(c):
            b = c % _NBUF
            pltpu.async_copy(
                table_hbm.at[idx_v.at[pl.ds(c * _CH, _CH)]], bufs[b], gsem[b])

        def gather_wait(c):
            b = c % _NBUF
            pltpu.make_async_copy(
                table_hbm.at[idx_v.at[pl.ds(c * _CH, _CH)]], bufs[b],
                gsem[b]).wait()

        def write_start(c):
            b = c % _NBUF
            pltpu.async_copy(
                bufs[b], out_hbm.at[pl.ds(base + c * _CH, _CH)], wsem[b])

        def write_wait(c):
            b = c % _NBUF
            pltpu.make_async_copy(
                bufs[b], out_hbm.at[pl.ds(base + c * _CH, _CH)],
                wsem[b]).wait()

        def scale_chunk(b):
            buf = bufs[b]

            def row(r, carry):
                for kk in range(DIM // 16):
                    sl = pl.ds(kk * 16, 16)
                    buf[r, sl] = buf[r, sl] * EMBED_SCALE
                return carry

            lax.fori_loop(0, _CH, row, 0)

        gather_start(0)
        if n_chunks > 1:
            gather_start(1)
        for c in range(n_chunks):
            gather_wait(c)
            scale_chunk(c % _NBUF)
            write_start(c)
            if c + 2 < n_chunks:
                if c >= 1:
                    write_wait(c - 1)
                gather_start(c + 2)
        write_wait(n_chunks - 2)
        write_wait(n_chunks - 1)

    return k(table, idx)


def _tc_add_pos(embed, scale, n_tok, seq_len):
    bs = 256
    n_seq_blocks = seq_len // bs
    n_batch = n_tok // seq_len

    def body(scale_ref, emb_ref, x_ref, pos_vmem):
        i = pl.program_id(0)
        j = pl.program_id(1)

        @pl.when(j == 0)
        def _():
            col = lax.broadcasted_iota(jnp.int32, (1, DIM), 1)
            is_cos = col >= HALF
            f = jnp.where(is_cos, col - HALF, col).astype(jnp.float32)
            inv_freq = jnp.exp(f * (-LN_THETA / HALF))

            @pl.when(i == 0)
            def _():
                p = lax.broadcasted_iota(
                    jnp.int32, (bs, DIM), 0).astype(jnp.float32)
                arg = p * inv_freq + jnp.where(is_cos, HALF_PI, 0.0)
                pos_vmem[...] = jnp.sin(arg) * scale_ref[0, 0]

            @pl.when(i > 0)
            def _():
                delta = bs * inv_freq[:, :HALF]
                cd = jnp.cos(delta)
                sd = jnp.sin(delta)
                s = pos_vmem[:, :HALF]
                c = pos_vmem[:, HALF:]
                pos_vmem[...] = jnp.concatenate(
                    [s * cd + c * sd, c * cd - s * sd], axis=1)

        x_ref[...] = emb_ref[...] + pos_vmem[...]

    grid = (n_seq_blocks, n_batch)
    blk = pl.BlockSpec((bs, DIM), lambda i, j: (j * n_seq_blocks + i, 0))
    return pl.pallas_call(
        body,
        grid=grid,
        in_specs=[
            pl.BlockSpec((1, 1), lambda i, j: (0, 0),
                         memory_space=pltpu.SMEM),
            blk,
        ],
        out_specs=blk,
        out_shape=jax.ShapeDtypeStruct((n_tok, DIM), jnp.float32),
        scratch_shapes=[pltpu.VMEM((bs, DIM), jnp.float32)],
    )(scale.reshape(1, 1), embed)


def kernel(src_tokens, table, scale):
    n_batch, seq_len = src_tokens.shape
    n_tok = n_batch * seq_len
    idx = src_tokens.reshape(-1).astype(jnp.int32)
    embed = _sc_gather_scale(table, idx, n_tok)
    x = _tc_add_pos(embed, scale, n_tok, seq_len)
    out_shape = (n_batch, seq_len, DIM)
    return (x.reshape(out_shape), embed.reshape(out_shape))


# speedup vs baseline: 1.2802x; 1.0103x over previous
import functools
import math

import jax
import jax.numpy as jnp
from jax import lax
from jax.experimental import pallas as pl
from jax.experimental.pallas import tpu as pltpu
from jax.experimental.pallas import tpu_sc as plsc

DIM = 1024
THETA = 2000.0
HALF = DIM // 2
EMBED_SCALE = math.sqrt(DIM)
LN_THETA = math.log(THETA)
HALF_PI = math.pi / 2.0

_SC_NC = 2
_SC_NS = 16
_SC_NW = _SC_NC * _SC_NS

_CH = 32
_NBUF = 3


def _sc_gather_scale(table, idx, n_tok):
    per_w = n_tok // _SC_NW
    n_chunks = per_w // _CH
    mesh = plsc.VectorSubcoreMesh(core_axis_name="c", subcore_axis_name="s")

    scratch = [pltpu.VMEM((per_w,), jnp.int32)]
    scratch += [pltpu.VMEM((_CH, DIM), jnp.float32) for _ in range(_NBUF)]
    scratch += [pltpu.SemaphoreType.DMA for _ in range(2 * _NBUF)]

    @functools.partial(
        pl.kernel,
        mesh=mesh,
        out_type=jax.ShapeDtypeStruct((n_tok, DIM), jnp.float32),
        scratch_types=scratch,
    )
    def k(table_hbm, idx_hbm, out_hbm, idx_v, *bufs_sems):
        bufs = bufs_sems[:_NBUF]
        gsem = bufs_sems[_NBUF:2 * _NBUF]
        wsem = bufs_sems[2 * _NBUF:]
        wid = lax.axis_index("s") * _SC_NC + lax.axis_index("c")
        base = wid * per_w
        pltpu.sync_copy(idx_hbm.at[pl.ds(base, per_w)], idx_v)

        def gather_start(c):
            b = c % _NBUF
            pltpu.async_copy(
                table_hbm.at[idx_v.at[pl.ds(c * _CH, _CH)]], bufs[b], gsem[b])

        def gather_wait(c):
            b = c % _NBUF
            pltpu.make_async_copy(
                table_hbm.at[idx_v.at[pl.ds(c * _CH, _CH)]], bufs[b],
                gsem[b]).wait()

        def write_start(c):
            b = c % _NBUF
            pltpu.async_copy(
                bufs[b], out_hbm.at[pl.ds(base + c * _CH, _CH)], wsem[b])

        def write_wait(c):
            b = c % _NBUF
            pltpu.make_async_copy(
                bufs[b], out_hbm.at[pl.ds(base + c * _CH, _CH)],
                wsem[b]).wait()

        def scale_chunk(b):
            buf = bufs[b]

            def row(r, carry):
                for kk in range(DIM // 16):
                    sl = pl.ds(kk * 16, 16)
                    buf[r, sl] = buf[r, sl] * EMBED_SCALE
                return carry

            lax.fori_loop(0, _CH, row, 0)

        gather_start(0)
        if n_chunks > 1:
            gather_start(1)
        for c in range(n_chunks):
            gather_wait(c)
            scale_chunk(c % _NBUF)
            write_start(c)
            if c + 2 < n_chunks:
                if c >= 1:
                    write_wait(c - 1)
                gather_start(c + 2)
        write_wait(n_chunks - 2)
        write_wait(n_chunks - 1)

    return k(table, idx)


def _tc_positions(scale, seq_len):
    bs = 256
    n_blocks = seq_len // bs

    def body(scale_ref, pos_ref):
        i = pl.program_id(0)
        col = lax.broadcasted_iota(jnp.int32, (bs, DIM), 1)
        is_cos = col >= HALF
        f = jnp.where(is_cos, col - HALF, col).astype(jnp.float32)
        inv_freq = jnp.exp(f * (-LN_THETA / HALF))
        row = lax.broadcasted_iota(jnp.int32, (bs, DIM), 0) + i * bs
        arg = row.astype(jnp.float32) * inv_freq
        arg = arg + jnp.where(is_cos, HALF_PI, 0.0)
        pos_ref[...] = jnp.sin(arg) * scale_ref[0, 0]

    return pl.pallas_call(
        body,
        grid=(n_blocks,),
        in_specs=[
            pl.BlockSpec((1, 1), lambda i: (0, 0), memory_space=pltpu.SMEM),
        ],
        out_specs=pl.BlockSpec((bs, DIM), lambda i: (i, 0)),
        out_shape=jax.ShapeDtypeStruct((seq_len, DIM), jnp.float32),
    )(scale.reshape(1, 1))


def _tc_add_pos(embed, pos, n_tok, seq_len):
    bs = 256
    n_seq_blocks = seq_len // bs
    n_batch = n_tok // seq_len

    def body(pos_ref, emb_ref, x_ref):
        x_ref[...] = emb_ref[...] + pos_ref[...]

    grid = (n_seq_blocks, n_batch)
    blk = pl.BlockSpec((bs, DIM), lambda i, j: (j * n_seq_blocks + i, 0))
    return pl.pallas_call(
        body,
        grid=grid,
        in_specs=[pl.BlockSpec((bs, DIM), lambda i, j: (i, 0)), blk],
        out_specs=blk,
        out_shape=jax.ShapeDtypeStruct((n_tok, DIM), jnp.float32),
    )(pos, embed)


def kernel(src_tokens, table, scale):
    n_batch, seq_len = src_tokens.shape
    n_tok = n_batch * seq_len
    idx = src_tokens.reshape(-1).astype(jnp.int32)
    pos = _tc_positions(scale, seq_len)
    embed = _sc_gather_scale(table, idx, n_tok)
    x = _tc_add_pos(embed, pos, n_tok, seq_len)
    out_shape = (n_batch, seq_len, DIM)
    return (x.reshape(out_shape), embed.reshape(out_shape))


# speedup vs baseline: 1.4257x; 1.1136x over previous
import functools
import math

import jax
import jax.numpy as jnp
from jax import lax
from jax.experimental import pallas as pl
from jax.experimental.pallas import tpu as pltpu
from jax.experimental.pallas import tpu_sc as plsc

DIM = 1024
THETA = 2000.0
HALF = DIM // 2
EMBED_SCALE = math.sqrt(DIM)
LN_THETA = math.log(THETA)
HALF_PI = math.pi / 2.0

_SC_NC = 2
_SC_NS = 16
_SC_NW = _SC_NC * _SC_NS

_CH = 32
_NBUF = 3


def _sc_gather_scale(table, idx, n_tok):
    per_w = n_tok // _SC_NW
    n_chunks = per_w // _CH
    mesh = plsc.VectorSubcoreMesh(core_axis_name="c", subcore_axis_name="s")

    scratch = [pltpu.VMEM((per_w,), jnp.int32)]
    scratch += [pltpu.VMEM((_CH, DIM), jnp.float32) for _ in range(_NBUF)]
    scratch += [pltpu.SemaphoreType.DMA for _ in range(2 * _NBUF)]

    @functools.partial(
        pl.kernel,
        mesh=mesh,
        out_type=jax.ShapeDtypeStruct((n_tok, DIM), jnp.float32),
        scratch_types=scratch,
    )
    def k(table_hbm, idx_hbm, out_hbm, idx_v, *bufs_sems):
        bufs = bufs_sems[:_NBUF]
        gsem = bufs_sems[_NBUF:2 * _NBUF]
        wsem = bufs_sems[2 * _NBUF:]
        wid = lax.axis_index("s") * _SC_NC + lax.axis_index("c")
        base = wid * per_w
        pltpu.sync_copy(idx_hbm.at[pl.ds(base, per_w)], idx_v)

        def gather_start(c):
            b = c % _NBUF
            pltpu.async_copy(
                table_hbm.at[idx_v.at[pl.ds(c * _CH, _CH)]], bufs[b], gsem[b])

        def gather_wait(c):
            b = c % _NBUF
            pltpu.make_async_copy(
                table_hbm.at[idx_v.at[pl.ds(c * _CH, _CH)]], bufs[b],
                gsem[b]).wait()

        def write_start(c):
            b = c % _NBUF
            pltpu.async_copy(
                bufs[b], out_hbm.at[pl.ds(base + c * _CH, _CH)], wsem[b])

        def write_wait(c):
            b = c % _NBUF
            pltpu.make_async_copy(
                bufs[b], out_hbm.at[pl.ds(base + c * _CH, _CH)],
                wsem[b]).wait()

        def scale_chunk(b):
            buf = bufs[b]

            def row(r, carry):
                for kk in range(DIM // 16):
                    sl = pl.ds(kk * 16, 16)
                    buf[r, sl] = buf[r, sl] * EMBED_SCALE
                return carry

            lax.fori_loop(0, _CH, row, 0)

        gather_start(0)
        if n_chunks > 1:
            gather_start(1)
        for c in range(n_chunks):
            gather_wait(c)
            scale_chunk(c % _NBUF)
            write_start(c)
            if c + 2 < n_chunks:
                if c >= 1:
                    write_wait(c - 1)
                gather_start(c + 2)
        write_wait(n_chunks - 2)
        write_wait(n_chunks - 1)

    return k(table, idx)


def _tc_positions(scale, seq_len):
    bs = 256
    n_blocks = seq_len // bs

    def body(scale_ref, pos_ref):
        i = pl.program_id(0)
        col = lax.broadcasted_iota(jnp.int32, (bs, DIM), 1)
        is_cos = col >= HALF
        f = jnp.where(is_cos, col - HALF, col).astype(jnp.float32)
        inv_freq = jnp.exp(f * (-LN_THETA / HALF))
        row = lax.broadcasted_iota(jnp.int32, (bs, DIM), 0) + i * bs
        arg = row.astype(jnp.float32) * inv_freq
        arg = arg + jnp.where(is_cos, HALF_PI, 0.0)
        pos_ref[...] = jnp.sin(arg) * scale_ref[0, 0]

    return pl.pallas_call(
        body,
        grid=(n_blocks,),
        in_specs=[
            pl.BlockSpec((1, 1), lambda i: (0, 0), memory_space=pltpu.SMEM),
        ],
        out_specs=pl.BlockSpec((bs, DIM), lambda i: (i, 0)),
        out_shape=jax.ShapeDtypeStruct((seq_len, DIM), jnp.float32),
    )(scale.reshape(1, 1))


def _tc_add_pos(embed, pos, n_tok, seq_len):
    bs = 512
    n_seq_blocks = seq_len // bs
    n_batch = n_tok // seq_len

    def body(pos_ref, emb_ref, x_ref):
        x_ref[...] = emb_ref[...] + pos_ref[...]

    grid = (n_seq_blocks, n_batch)
    blk = pl.BlockSpec((bs, DIM), lambda i, j: (j * n_seq_blocks + i, 0))
    return pl.pallas_call(
        body,
        grid=grid,
        in_specs=[pl.BlockSpec((bs, DIM), lambda i, j: (i, 0)), blk],
        out_specs=blk,
        out_shape=jax.ShapeDtypeStruct((n_tok, DIM), jnp.float32),
    )(pos, embed)


def kernel(src_tokens, table, scale):
    n_batch, seq_len = src_tokens.shape
    n_tok = n_batch * seq_len
    idx = src_tokens.reshape(-1).astype(jnp.int32)
    pos = _tc_positions(scale, seq_len)
    embed = _sc_gather_scale(table, idx, n_tok)
    x = _tc_add_pos(embed, pos, n_tok, seq_len)
    out_shape = (n_batch, seq_len, DIM)
    return (x.reshape(out_shape), embed.reshape(out_shape))


# speedup vs baseline: 1.4703x; 1.0313x over previous
import functools
import math

import jax
import jax.numpy as jnp
from jax import lax
from jax.experimental import pallas as pl
from jax.experimental.pallas import tpu as pltpu
from jax.experimental.pallas import tpu_sc as plsc

DIM = 1024
THETA = 2000.0
HALF = DIM // 2
EMBED_SCALE = math.sqrt(DIM)
LN_THETA = math.log(THETA)
HALF_PI = math.pi / 2.0

_SC_NC = 2
_SC_NS = 16
_SC_NW = _SC_NC * _SC_NS

_CH = 32
_NBUF = 3


def _sc_gather_scale(table, idx, n_tok):
    per_w = n_tok // _SC_NW
    n_chunks = per_w // _CH
    mesh = plsc.VectorSubcoreMesh(core_axis_name="c", subcore_axis_name="s")

    scratch = [pltpu.VMEM((per_w,), jnp.int32)]
    scratch += [pltpu.VMEM((_CH, DIM), jnp.float32) for _ in range(_NBUF)]
    scratch += [pltpu.SemaphoreType.DMA for _ in range(2 * _NBUF)]

    @functools.partial(
        pl.kernel,
        mesh=mesh,
        out_type=jax.ShapeDtypeStruct((n_tok, DIM), jnp.float32),
        scratch_types=scratch,
    )
    def k(table_hbm, idx_hbm, out_hbm, idx_v, *bufs_sems):
        bufs = bufs_sems[:_NBUF]
        gsem = bufs_sems[_NBUF:2 * _NBUF]
        wsem = bufs_sems[2 * _NBUF:]
        wid = lax.axis_index("s") * _SC_NC + lax.axis_index("c")
        base = wid * per_w
        pltpu.sync_copy(idx_hbm.at[pl.ds(base, per_w)], idx_v)

        def gather_start(c):
            b = c % _NBUF
            pltpu.async_copy(
                table_hbm.at[idx_v.at[pl.ds(c * _CH, _CH)]], bufs[b], gsem[b])

        def gather_wait(c):
            b = c % _NBUF
            pltpu.make_async_copy(
                table_hbm.at[idx_v.at[pl.ds(c * _CH, _CH)]], bufs[b],
                gsem[b]).wait()

        def write_start(c):
            b = c % _NBUF
            pltpu.async_copy(
                bufs[b], out_hbm.at[pl.ds(base + c * _CH, _CH)], wsem[b])

        def write_wait(c):
            b = c % _NBUF
            pltpu.make_async_copy(
                bufs[b], out_hbm.at[pl.ds(base + c * _CH, _CH)],
                wsem[b]).wait()

        def scale_chunk(b):
            buf = bufs[b]

            def row(r, carry):
                for kk in range(DIM // 16):
                    sl = pl.ds(kk * 16, 16)
                    buf[r, sl] = buf[r, sl] * EMBED_SCALE
                return carry

            lax.fori_loop(0, _CH, row, 0)

        gather_start(0)
        if n_chunks > 1:
            gather_start(1)
        for c in range(n_chunks):
            gather_wait(c)
            scale_chunk(c % _NBUF)
            write_start(c)
            if c + 2 < n_chunks:
                if c >= 1:
                    write_wait(c - 1)
                gather_start(c + 2)
        write_wait(n_chunks - 2)
        write_wait(n_chunks - 1)

    return k(table, idx)


def _tc_positions(scale, seq_len):
    bs = 256
    n_blocks = seq_len // bs

    def body(scale_ref, pos_ref):
        i = pl.program_id(0)
        col = lax.broadcasted_iota(jnp.int32, (bs, DIM), 1)
        is_cos = col >= HALF
        f = jnp.where(is_cos, col - HALF, col).astype(jnp.float32)
        inv_freq = jnp.exp(f * (-LN_THETA / HALF))
        row = lax.broadcasted_iota(jnp.int32, (bs, DIM), 0) + i * bs
        arg = row.astype(jnp.float32) * inv_freq
        arg = arg + jnp.where(is_cos, HALF_PI, 0.0)
        pos_ref[...] = jnp.sin(arg) * scale_ref[0, 0]

    return pl.pallas_call(
        body,
        grid=(n_blocks,),
        in_specs=[
            pl.BlockSpec((1, 1), lambda i: (0, 0), memory_space=pltpu.SMEM),
        ],
        out_specs=pl.BlockSpec((bs, DIM), lambda i: (i, 0)),
        out_shape=jax.ShapeDtypeStruct((seq_len, DIM), jnp.float32),
    )(scale.reshape(1, 1))


def _tc_add_pos(embed, pos, n_tok, seq_len):
    bs = 1024
    n_seq_blocks = seq_len // bs
    n_batch = n_tok // seq_len

    def body(pos_ref, emb_ref, x_ref):
        x_ref[...] = emb_ref[...] + pos_ref[...]

    grid = (n_seq_blocks, n_batch)
    blk = pl.BlockSpec((bs, DIM), lambda i, j: (j * n_seq_blocks + i, 0))
    return pl.pallas_call(
        body,
        grid=grid,
        in_specs=[pl.BlockSpec((bs, DIM), lambda i, j: (i, 0)), blk],
        out_specs=blk,
        out_shape=jax.ShapeDtypeStruct((n_tok, DIM), jnp.float32),
    )(pos, embed)


def kernel(src_tokens, table, scale):
    n_batch, seq_len = src_tokens.shape
    n_tok = n_batch * seq_len
    idx = src_tokens.reshape(-1).astype(jnp.int32)
    pos = _tc_positions(scale, seq_len)
    embed = _sc_gather_scale(table, idx, n_tok)
    x = _tc_add_pos(embed, pos, n_tok, seq_len)
    out_shape = (n_batch, seq_len, DIM)
    return (x.reshape(out_shape), embed.reshape(out_shape))


# speedup vs baseline: 1.5073x; 1.0251x over previous
import functools
import math

import jax
import jax.numpy as jnp
from jax import lax
from jax.experimental import pallas as pl
from jax.experimental.pallas import tpu as pltpu
from jax.experimental.pallas import tpu_sc as plsc

DIM = 1024
THETA = 2000.0
HALF = DIM // 2
EMBED_SCALE = math.sqrt(DIM)
LN_THETA = math.log(THETA)
HALF_PI = math.pi / 2.0

_SC_NC = 2
_SC_NS = 16
_SC_NW = _SC_NC * _SC_NS

_CH = 32
_NBUF = 3


def _sc_gather_scale(table, idx, n_tok):
    per_w = n_tok // _SC_NW
    n_chunks = per_w // _CH
    mesh = plsc.VectorSubcoreMesh(core_axis_name="c", subcore_axis_name="s")

    scratch = [pltpu.VMEM((per_w,), jnp.int32)]
    scratch += [pltpu.VMEM((_CH, DIM), jnp.float32) for _ in range(_NBUF)]
    scratch += [pltpu.SemaphoreType.DMA for _ in range(2 * _NBUF)]

    @functools.partial(
        pl.kernel,
        mesh=mesh,
        out_type=jax.ShapeDtypeStruct((n_tok, DIM), jnp.float32),
        scratch_types=scratch,
    )
    def k(table_hbm, idx_hbm, out_hbm, idx_v, *bufs_sems):
        bufs = bufs_sems[:_NBUF]
        gsem = bufs_sems[_NBUF:2 * _NBUF]
        wsem = bufs_sems[2 * _NBUF:]
        wid = lax.axis_index("s") * _SC_NC + lax.axis_index("c")
        base = wid * per_w
        pltpu.sync_copy(idx_hbm.at[pl.ds(base, per_w)], idx_v)

        def gather_start(c):
            b = c % _NBUF
            pltpu.async_copy(
                table_hbm.at[idx_v.at[pl.ds(c * _CH, _CH)]], bufs[b], gsem[b])

        def gather_wait(c):
            b = c % _NBUF
            pltpu.make_async_copy(
                table_hbm.at[idx_v.at[pl.ds(c * _CH, _CH)]], bufs[b],
                gsem[b]).wait()

        def write_start(c):
            b = c % _NBUF
            pltpu.async_copy(
                bufs[b], out_hbm.at[pl.ds(base + c * _CH, _CH)], wsem[b])

        def write_wait(c):
            b = c % _NBUF
            pltpu.make_async_copy(
                bufs[b], out_hbm.at[pl.ds(base + c * _CH, _CH)],
                wsem[b]).wait()

        def scale_chunk(b):
            buf = bufs[b]

            def row(r, carry):
                for kk in range(DIM // 16):
                    sl = pl.ds(kk * 16, 16)
                    buf[r, sl] = buf[r, sl] * EMBED_SCALE
                return carry

            lax.fori_loop(0, _CH, row, 0)

        gather_start(0)
        if n_chunks > 1:
            gather_start(1)
        for c in range(n_chunks):
            gather_wait(c)
            scale_chunk(c % _NBUF)
            write_start(c)
            if c + 2 < n_chunks:
                if c >= 1:
                    write_wait(c - 1)
                gather_start(c + 2)
        write_wait(n_chunks - 2)
        write_wait(n_chunks - 1)

    return k(table, idx)


def _tc_positions(scale, seq_len):
    bs = 256
    n_blocks = seq_len // bs

    def body(scale_ref, pos_ref):
        i = pl.program_id(0)
        col = lax.broadcasted_iota(jnp.int32, (bs, DIM), 1)
        is_cos = col >= HALF
        f = jnp.where(is_cos, col - HALF, col).astype(jnp.float32)
        inv_freq = jnp.exp(f * (-LN_THETA / HALF))
        row = lax.broadcasted_iota(jnp.int32, (bs, DIM), 0) + i * bs
        arg = row.astype(jnp.float32) * inv_freq
        arg = arg + jnp.where(is_cos, HALF_PI, 0.0)
        pos_ref[...] = jnp.sin(arg) * scale_ref[0, 0]

    return pl.pallas_call(
        body,
        grid=(n_blocks,),
        in_specs=[
            pl.BlockSpec((1, 1), lambda i: (0, 0), memory_space=pltpu.SMEM),
        ],
        out_specs=pl.BlockSpec((bs, DIM), lambda i: (i, 0)),
        out_shape=jax.ShapeDtypeStruct((seq_len, DIM), jnp.float32),
    )(scale.reshape(1, 1))


def _tc_add_pos(embed, pos, n_tok, seq_len):
    bs = 2048
    n_seq_blocks = seq_len // bs
    n_batch = n_tok // seq_len

    def body(pos_ref, emb_ref, x_ref):
        x_ref[...] = emb_ref[...] + pos_ref[...]

    grid = (n_seq_blocks, n_batch)
    blk = pl.BlockSpec((bs, DIM), lambda i, j: (j * n_seq_blocks + i, 0))
    return pl.pallas_call(
        body,
        grid=grid,
        in_specs=[pl.BlockSpec((bs, DIM), lambda i, j: (i, 0)), blk],
        out_specs=blk,
        out_shape=jax.ShapeDtypeStruct((n_tok, DIM), jnp.float32),
    )(pos, embed)


def kernel(src_tokens, table, scale):
    n_batch, seq_len = src_tokens.shape
    n_tok = n_batch * seq_len
    idx = src_tokens.reshape(-1).astype(jnp.int32)
    pos = _tc_positions(scale, seq_len)
    embed = _sc_gather_scale(table, idx, n_tok)
    x = _tc_add_pos(embed, pos, n_tok, seq_len)
    out_shape = (n_batch, seq_len, DIM)
    return (x.reshape(out_shape), embed.reshape(out_shape))


# speedup vs baseline: 1.5240x; 1.0111x over previous
import functools
import math

import jax
import jax.numpy as jnp
from jax import lax
from jax.experimental import pallas as pl
from jax.experimental.pallas import tpu as pltpu
from jax.experimental.pallas import tpu_sc as plsc

DIM = 1024
THETA = 2000.0
HALF = DIM // 2
EMBED_SCALE = math.sqrt(DIM)
LN_THETA = math.log(THETA)
HALF_PI = math.pi / 2.0

_SC_NC = 2
_SC_NS = 16
_SC_NW = _SC_NC * _SC_NS

_CH = 32
_NBUF = 3


def _sc_gather_scale(table, idx, n_tok):
    per_w = n_tok // _SC_NW
    n_chunks = per_w // _CH
    mesh = plsc.VectorSubcoreMesh(core_axis_name="c", subcore_axis_name="s")

    scratch = [pltpu.VMEM((per_w,), jnp.int32)]
    scratch += [pltpu.VMEM((_CH, DIM), jnp.float32) for _ in range(_NBUF)]
    scratch += [pltpu.SemaphoreType.DMA for _ in range(2 * _NBUF)]

    @functools.partial(
        pl.kernel,
        mesh=mesh,
        out_type=jax.ShapeDtypeStruct((n_tok, DIM), jnp.float32),
        scratch_types=scratch,
    )
    def k(table_hbm, idx_hbm, out_hbm, idx_v, *bufs_sems):
        bufs = bufs_sems[:_NBUF]
        gsem = bufs_sems[_NBUF:2 * _NBUF]
        wsem = bufs_sems[2 * _NBUF:]
        wid = lax.axis_index("s") * _SC_NC + lax.axis_index("c")
        base = wid * per_w
        pltpu.sync_copy(idx_hbm.at[pl.ds(base, per_w)], idx_v)

        def gather_start(c):
            b = c % _NBUF
            pltpu.async_copy(
                table_hbm.at[idx_v.at[pl.ds(c * _CH, _CH)]], bufs[b], gsem[b])

        def gather_wait(c):
            b = c % _NBUF
            pltpu.make_async_copy(
                table_hbm.at[idx_v.at[pl.ds(c * _CH, _CH)]], bufs[b],
                gsem[b]).wait()

        def write_start(c):
            b = c % _NBUF
            pltpu.async_copy(
                bufs[b], out_hbm.at[pl.ds(base + c * _CH, _CH)], wsem[b])

        def write_wait(c):
            b = c % _NBUF
            pltpu.make_async_copy(
                bufs[b], out_hbm.at[pl.ds(base + c * _CH, _CH)],
                wsem[b]).wait()

        def scale_chunk(b):
            buf = bufs[b]

            def row(r, carry):
                for kk in range(DIM // 16):
                    sl = pl.ds(kk * 16, 16)
                    buf[r, sl] = buf[r, sl] * EMBED_SCALE
                return carry

            lax.fori_loop(0, _CH, row, 0)

        gather_start(0)
        if n_chunks > 1:
            gather_start(1)
        for c in range(n_chunks):
            gather_wait(c)
            scale_chunk(c % _NBUF)
            write_start(c)
            if c + 2 < n_chunks:
                if c >= 1:
                    write_wait(c - 1)
                gather_start(c + 2)
        write_wait(n_chunks - 2)
        write_wait(n_chunks - 1)

    return k(table, idx)


_PB = 256


def _tc_add_pos(embed, scale, n_tok, seq_len):
    n_batch = n_tok // seq_len
    n_blocks = seq_len // _PB

    def body(scale_ref, emb_ref, x_ref, pos_vmem):
        j = pl.program_id(0)

        @pl.when(j == 0)
        def _():
            s = scale_ref[0, 0]
            col = lax.broadcasted_iota(jnp.int32, (_PB, HALF), 1)
            w = jnp.exp(col.astype(jnp.float32) * (-LN_THETA / HALF))
            brow = lax.broadcasted_iota(
                jnp.int32, (_PB, HALF), 0).astype(jnp.float32)
            sb = jnp.sin(brow * w) * s
            cb = jnp.cos(brow * w) * s
            pos_vmem[0:_PB, :HALF] = sb
            pos_vmem[0:_PB, HALF:] = cb
            for a in range(1, n_blocks):
                wa = w[0:1, :] * float(_PB * a)
                sa = jnp.sin(wa)
                ca = jnp.cos(wa)
                lo = a * _PB
                pos_vmem[lo:lo + _PB, :HALF] = sa * cb + ca * sb
                pos_vmem[lo:lo + _PB, HALF:] = ca * cb - sa * sb

        x_ref[...] = emb_ref[...] + pos_vmem[...]

    blk = pl.BlockSpec((seq_len, DIM), lambda j: (j, 0))
    return pl.pallas_call(
        body,
        grid=(n_batch,),
        in_specs=[
            pl.BlockSpec((1, 1), lambda j: (0, 0), memory_space=pltpu.SMEM),
            blk,
        ],
        out_specs=blk,
        out_shape=jax.ShapeDtypeStruct((n_tok, DIM), jnp.float32),
        scratch_shapes=[pltpu.VMEM((seq_len, DIM), jnp.float32)],
    )(scale.reshape(1, 1), embed)


def kernel(src_tokens, table, scale):
    n_batch, seq_len = src_tokens.shape
    n_tok = n_batch * seq_len
    idx = src_tokens.reshape(-1).astype(jnp.int32)
    embed = _sc_gather_scale(table, idx, n_tok)
    x = _tc_add_pos(embed, scale, n_tok, seq_len)
    out_shape = (n_batch, seq_len, DIM)
    return (x.reshape(out_shape), embed.reshape(out_shape))


# speedup vs baseline: 1.5284x; 1.0029x over previous
import functools
import math

import jax
import jax.numpy as jnp
from jax import lax
from jax.experimental import pallas as pl
from jax.experimental.pallas import tpu as pltpu
from jax.experimental.pallas import tpu_sc as plsc

DIM = 1024
THETA = 2000.0
HALF = DIM // 2
EMBED_SCALE = math.sqrt(DIM)
LN_THETA = math.log(THETA)

_SC_NC = 2
_SC_NS = 16
_SC_NW = _SC_NC * _SC_NS

_CH = 32
_NBUF = 3


def _sc_gather_scale(table, idx, n_tok):
    per_w = n_tok // _SC_NW
    n_chunks = per_w // _CH
    mesh = plsc.VectorSubcoreMesh(core_axis_name="c", subcore_axis_name="s")

    scratch = [pltpu.VMEM((per_w,), jnp.int32)]
    scratch += [pltpu.VMEM((_CH, DIM), jnp.float32) for _ in range(_NBUF)]
    scratch += [pltpu.SemaphoreType.DMA for _ in range(2 * _NBUF)]

    @functools.partial(
        pl.kernel,
        mesh=mesh,
        out_type=jax.ShapeDtypeStruct((n_tok, DIM), jnp.float32),
        scratch_types=scratch,
    )
    def k(table_hbm, idx_hbm, out_hbm, idx_v, *bufs_sems):
        bufs = bufs_sems[:_NBUF]
        gsem = bufs_sems[_NBUF:2 * _NBUF]
        wsem = bufs_sems[2 * _NBUF:]
        wid = lax.axis_index("s") * _SC_NC + lax.axis_index("c")
        base = wid * per_w
        pltpu.sync_copy(idx_hbm.at[pl.ds(base, per_w)], idx_v)

        def gather_start(c):
            b = c % _NBUF
            pltpu.async_copy(
                table_hbm.at[idx_v.at[pl.ds(c * _CH, _CH)]], bufs[b], gsem[b])

        def gather_wait(c):
            b = c % _NBUF
            pltpu.make_async_copy(
                table_hbm.at[idx_v.at[pl.ds(c * _CH, _CH)]], bufs[b],
                gsem[b]).wait()

        def write_start(c):
            b = c % _NBUF
            pltpu.async_copy(
                bufs[b], out_hbm.at[pl.ds(base + c * _CH, _CH)], wsem[b])

        def write_wait(c):
            b = c % _NBUF
            pltpu.make_async_copy(
                bufs[b], out_hbm.at[pl.ds(base + c * _CH, _CH)],
                wsem[b]).wait()

        def scale_chunk(b):
            buf = bufs[b]

            def row(r, carry):
                for kk in range(DIM // 16):
                    sl = pl.ds(kk * 16, 16)
                    buf[r, sl] = buf[r, sl] * EMBED_SCALE
                return carry

            lax.fori_loop(0, _CH, row, 0)

        gather_start(0)
        if n_chunks > 1:
            gather_start(1)
        for c in range(n_chunks):
            gather_wait(c)
            scale_chunk(c % _NBUF)
            write_start(c)
            if c + 2 < n_chunks:
                if c >= 1:
                    write_wait(c - 1)
                gather_start(c + 2)
        write_wait(n_chunks - 2)
        write_wait(n_chunks - 1)

    return k(table, idx)


_PB = 256


def _tc_add_pos(embed, scale, n_tok, seq_len):
    n_batch = n_tok // seq_len
    n_blocks = seq_len // _PB

    def body(scale_ref, emb_ref, x_ref, pos_vmem):
        j = pl.program_id(0)

        @pl.when(j == 0)
        def _():
            s = scale_ref[0, 0]
            col = lax.broadcasted_iota(jnp.int32, (_PB, HALF), 1)
            w = jnp.exp(col.astype(jnp.float32) * (-LN_THETA / HALF))
            brow = lax.broadcasted_iota(
                jnp.int32, (_PB, HALF), 0).astype(jnp.float32)
            sb = jnp.sin(brow * w) * s
            cb = jnp.cos(brow * w) * s
            pos_vmem[0:_PB, :HALF] = sb
            pos_vmem[0:_PB, HALF:] = cb
            for a in range(1, n_blocks):
                wa = w[0:1, :] * float(_PB * a)
                sa = jnp.sin(wa)
                ca = jnp.cos(wa)
                lo = a * _PB
                pos_vmem[lo:lo + _PB, :HALF] = sa * cb + ca * sb
                pos_vmem[lo:lo + _PB, HALF:] = ca * cb - sa * sb

        x_ref[...] = emb_ref[...] + pos_vmem[...]

    blk = pl.BlockSpec((seq_len, DIM), lambda j: (j, 0))
    return pl.pallas_call(
        body,
        grid=(n_batch,),
        in_specs=[
            pl.BlockSpec((1, 1), lambda j: (0, 0), memory_space=pltpu.SMEM),
            blk,
        ],
        out_specs=blk,
        out_shape=jax.ShapeDtypeStruct((n_tok, DIM), jnp.float32),
        scratch_shapes=[pltpu.VMEM((seq_len, DIM), jnp.float32)],
    )(scale.reshape(1, 1), embed)


def kernel(src_tokens, table, scale):
    n_batch, seq_len = src_tokens.shape
    n_tok = n_batch * seq_len
    idx = src_tokens.reshape(-1).astype(jnp.int32)
    embed = _sc_gather_scale(table, idx, n_tok)
    x = _tc_add_pos(embed, scale, n_tok, seq_len)
    out_shape = (n_batch, seq_len, DIM)
    return (x.reshape(out_shape), embed.reshape(out_shape))
